# baseline (speedup 1.0000x reference)
import jax
import jax.numpy as jnp
from jax.experimental import pallas as pl
from jax.experimental.pallas import tpu as pltpu

_WIN = 32
_K = 8


def _mc_kernel(vis_ref, q_ref, w1_ref, w2_ref, out_ref, scratch_ref):
    s = pl.program_id(1)
    T = vis_ref.shape[1]
    C = vis_ref.shape[2]
    L = q_ref.shape[1]
    M = T - 3
    NW = T - _WIN + 1

    @pl.when(s == 0)
    def _compute():
        vis = vis_ref[0]
        q = q_ref[0]
        e1 = jnp.dot(q, w1_ref[...], preferred_element_type=jnp.float32)
        e2 = jnp.dot(q, w2_ref[...], preferred_element_type=jnp.float32)
        sim = jax.lax.dot_general(
            vis, e1, (((1,), (1,)), ((), ())),
            preferred_element_type=jnp.float32)
        mx = jnp.max(sim, axis=1, keepdims=True)
        li = jax.lax.broadcasted_iota(jnp.int32, (T, L), 1)
        labels = jnp.min(jnp.where(sim == mx, li, L), axis=1, keepdims=True)

        l0 = labels[0:M]
        l1 = labels[1:M + 1]
        l2 = labels[2:M + 2]
        l3 = labels[3:M + 3]

        def _cnt(a, b):
            return (a == b).astype(jnp.int32)

        c0 = 1 + _cnt(l0, l1) + _cnt(l0, l2) + _cnt(l0, l3)
        c1 = 1 + _cnt(l1, l0) + _cnt(l1, l2) + _cnt(l1, l3)
        c2 = 1 + _cnt(l2, l0) + _cnt(l2, l1) + _cnt(l2, l3)
        c3 = 1 + _cnt(l3, l0) + _cnt(l3, l1) + _cnt(l3, l2)
        s = jnp.maximum(
            jnp.maximum(c0 * 32 - l0, c1 * 32 - l1),
            jnp.maximum(c2 * 32 - l2, c3 * 32 - l3))
        maj = (-s) & 31

        oh = (maj == jax.lax.broadcasted_iota(jnp.int32, (M, 32), 1))
        e2p = jnp.concatenate(
            [e2, jnp.zeros((32 - L, C), jnp.float32)], axis=0)
        modrows = jnp.dot(oh.astype(jnp.float32), e2p,
                          preferred_element_type=jnp.float32)
        avg4 = (vis[0:M] + vis[1:M + 1] + vis[2:M + 2] + vis[3:M + 3]) * 0.25
        scratch_ref[0:M, :] = modrows * avg4

    for half in range(2):
        @pl.when(s == half)
        def _emit(half=half):
            for j in range(_K // 2):
                k = half * (_K // 2) + j
                out_ref[0, :, j * C:(j + 1) * C] = (
                    scratch_ref[4 * k:4 * k + NW, :])


def kernel(vis_feats, query, W1, W2):
    B, T, C = vis_feats.shape
    L = query.shape[1]
    NW = T - _WIN + 1
    out3 = pl.pallas_call(
        _mc_kernel,
        grid=(B, 2),
        in_specs=[
            pl.BlockSpec((1, T, C), lambda b, s: (b, 0, 0)),
            pl.BlockSpec((1, L, C), lambda b, s: (b, 0, 0)),
            pl.BlockSpec((C, C), lambda b, s: (0, 0)),
            pl.BlockSpec((C, C), lambda b, s: (0, 0)),
        ],
        out_specs=pl.BlockSpec((1, NW, _K * C // 2), lambda b, s: (b, 0, s)),
        out_shape=jax.ShapeDtypeStruct((B, NW, _K * C), jnp.float32),
        scratch_shapes=[pltpu.VMEM((T, C), jnp.float32)],
    )(vis_feats, query, W1, W2)
    return out3.reshape(B, NW, _K, C)


# speedup vs baseline: 1.2342x; 1.2342x over previous
import jax
import jax.numpy as jnp
from jax.experimental import pallas as pl
from jax.experimental.pallas import tpu as pltpu

_WIN = 32
_K = 8
_D = 4


def _row_chunks(nw):
    base = (nw // _D) & ~7
    starts = [i * base for i in range(_D)]
    lens = [base] * (_D - 1) + [nw - base * (_D - 1)]
    return list(zip(starts, lens))


def _mc_kernel(vis_ref, q_ref, w1_ref, w2_ref, out_hbm, obuf, sems):
    b = pl.program_id(0)
    nb = pl.num_programs(0)
    T = vis_ref.shape[1]
    C = vis_ref.shape[2]
    L = q_ref.shape[1]
    M = T - 3
    NW = T - _WIN + 1
    chunks = _row_chunks(NW)
    buf = jax.lax.rem(b, 2)

    def _copies(src_slot, dst_b):
        ops = []
        for j, (r0, rn) in enumerate(chunks):
            ops.append(pltpu.make_async_copy(
                obuf.at[src_slot, pl.ds(r0, rn), :],
                out_hbm.at[dst_b, pl.ds(r0, rn), :],
                sems.at[src_slot, j]))
        return ops

    @pl.when(b >= 2)
    def _wait_prev():
        for op in _copies(buf, b - 2):
            op.wait()

    vis = vis_ref[0]
    q = q_ref[0]
    e1 = jnp.dot(q, w1_ref[...], preferred_element_type=jnp.float32)
    e2 = jnp.dot(q, w2_ref[...], preferred_element_type=jnp.float32)
    sim = jax.lax.dot_general(
        vis, e1, (((1,), (1,)), ((), ())),
        preferred_element_type=jnp.float32)
    mx = jnp.max(sim, axis=1, keepdims=True)
    li = jax.lax.broadcasted_iota(jnp.int32, (T, L), 1)
    labels = jnp.min(jnp.where(sim == mx, li, L), axis=1, keepdims=True)

    l0 = labels[0:M]
    l1 = labels[1:M + 1]
    l2 = labels[2:M + 2]
    l3 = labels[3:M + 3]

    def _cnt(a, c):
        return (a == c).astype(jnp.int32)

    c0 = 1 + _cnt(l0, l1) + _cnt(l0, l2) + _cnt(l0, l3)
    c1 = 1 + _cnt(l1, l0) + _cnt(l1, l2) + _cnt(l1, l3)
    c2 = 1 + _cnt(l2, l0) + _cnt(l2, l1) + _cnt(l2, l3)
    c3 = 1 + _cnt(l3, l0) + _cnt(l3, l1) + _cnt(l3, l2)
    s = jnp.maximum(
        jnp.maximum(c0 * 32 - l0, c1 * 32 - l1),
        jnp.maximum(c2 * 32 - l2, c3 * 32 - l3))
    maj = (-s) & 31

    oh = (maj == jax.lax.broadcasted_iota(jnp.int32, (M, 32), 1))
    e2p = jnp.concatenate(
        [e2, jnp.zeros((32 - L, C), jnp.float32)], axis=0)
    modrows = jnp.dot(oh.astype(jnp.float32), e2p,
                      preferred_element_type=jnp.float32)
    avg4 = (vis[0:M] + vis[1:M + 1] + vis[2:M + 2] + vis[3:M + 3]) * 0.25
    modavg = modrows * avg4

    for k in range(_K):
        obuf[buf, :, k * C:(k + 1) * C] = modavg[4 * k:4 * k + NW, :]

    for op in _copies(buf, b):
        op.start()

    @pl.when(b == nb - 1)
    def _drain():
        for op in _copies(buf, b):
            op.wait()

        @pl.when(nb > 1)
        def _drain_other():
            for op in _copies(1 - buf, b - 1):
                op.wait()


def kernel(vis_feats, query, W1, W2):
    B, T, C = vis_feats.shape
    L = query.shape[1]
    NW = T - _WIN + 1
    out3 = pl.pallas_call(
        _mc_kernel,
        grid=(B,),
        in_specs=[
            pl.BlockSpec((1, T, C), lambda b: (b, 0, 0)),
            pl.BlockSpec((1, L, C), lambda b: (b, 0, 0)),
            pl.BlockSpec((C, C), lambda b: (0, 0)),
            pl.BlockSpec((C, C), lambda b: (0, 0)),
        ],
        out_specs=pl.BlockSpec(memory_space=pl.ANY),
        out_shape=jax.ShapeDtypeStruct((B, NW, _K * C), jnp.float32),
        scratch_shapes=[
            pltpu.VMEM((2, NW, _K * C), jnp.float32),
            pltpu.SemaphoreType.DMA((2, _D)),
        ],
    )(vis_feats, query, W1, W2)
    return out3.reshape(B, NW, _K, C)
